# baseline (speedup 1.0000x reference)
import functools

import jax
import jax.numpy as jnp
from jax import lax
from jax.experimental import pallas as pl
from jax.experimental.pallas import tpu as pltpu
from jax.experimental.pallas import tpu_sc as plsc

N = 10000
E = 320000
DF = 128
DE = 4
L = 128

NC = 2
NS = 16
NW = NC * NS
EPW = E // NW
C = 40
NCHUNK = EPW // C
RPS = 624

_f32 = jnp.float32


def _dot(a, b):
    return jnp.dot(a, b, preferred_element_type=_f32)


def _dot16(a, b):
    return jnp.dot(a.astype(jnp.bfloat16), b.astype(jnp.bfloat16),
                   preferred_element_type=_f32)


def _node_encode_body(x_ref, wne_ref, bne_ref, wa_ref, wb_ref,
                      h_ref, p_ref, q_ref):
    h = jnp.maximum(_dot(x_ref[...], wne_ref[...]) + bne_ref[...], 0.0)
    h_ref[...] = h
    p_ref[...] = _dot(h, wa_ref[...])
    q_ref[...] = _dot(h, wb_ref[...])


def _node_encode(x, W_ne, b_ne, Wa, Wb):
    return pl.pallas_call(
        _node_encode_body,
        out_shape=[jax.ShapeDtypeStruct((N, L), _f32)] * 3,
    )(x, W_ne, b_ne, Wa, Wb)


BE = 6400


def _pack_pairs(u):
    r = u.shape[0]
    u16 = jax.lax.bitcast_convert_type(u.astype(jnp.bfloat16), jnp.int16)
    u3 = u16.reshape(r // 2, 2, L).astype(jnp.int32)
    lo = jnp.bitwise_and(u3[:, 0, :], jnp.int32(0xFFFF))
    hi = jax.lax.shift_left(u3[:, 1, :], jnp.int32(16))
    return jnp.bitwise_or(lo, hi)


def _edge_u0_body(ea_ref, wee_ref, bee_ref, wc_ref, be_ref, u_ref):
    e0 = jnp.maximum(_dot(ea_ref[...], wee_ref[...]) + bee_ref[...], 0.0)
    u_ref[...] = _pack_pairs(_dot16(e0, wc_ref[...]) + be_ref[...])


def _edge_u0(edge_attr, W_ee, b_ee, Wc, be):
    return pl.pallas_call(
        _edge_u0_body,
        grid=(E // BE,),
        in_specs=[
            pl.BlockSpec((BE, DE), lambda i: (i, 0)),
            pl.BlockSpec((DE, L), lambda i: (0, 0)),
            pl.BlockSpec((1, L), lambda i: (0, 0)),
            pl.BlockSpec((L, L), lambda i: (0, 0)),
            pl.BlockSpec((1, L), lambda i: (0, 0)),
        ],
        out_specs=pl.BlockSpec((BE // 2, L), lambda i: (i, 0)),
        out_shape=jax.ShapeDtypeStruct((E // 2, L), jnp.int32),
    )(edge_attr, W_ee, b_ee, Wc, be)


def _edge_u_body(e_ref, wc_ref, be_ref, u_ref):
    u_ref[...] = _pack_pairs(_dot16(e_ref[...], wc_ref[...]) + be_ref[...])


def _edge_u(e, Wc, be):
    return pl.pallas_call(
        _edge_u_body,
        grid=(E // BE,),
        in_specs=[
            pl.BlockSpec((BE, L), lambda i: (i, 0)),
            pl.BlockSpec((L, L), lambda i: (0, 0)),
            pl.BlockSpec((1, L), lambda i: (0, 0)),
        ],
        out_specs=pl.BlockSpec((BE // 2, L), lambda i: (i, 0)),
        out_shape=jax.ShapeDtypeStruct((E // 2, L), jnp.int32),
    )(e, Wc, be)


def _node_update_body(h_ref, a_ref, wnh_ref, wna_ref, bn_ref,
                      wa_ref, wb_ref, h1_ref, p_ref, q_ref):
    agg = a_ref[0] + a_ref[1]
    h1 = jnp.maximum(
        _dot(h_ref[...], wnh_ref[...]) + _dot(agg, wna_ref[...]) + bn_ref[...],
        0.0)
    h1_ref[...] = h1
    p_ref[...] = _dot(h1, wa_ref[...])
    q_ref[...] = _dot(h1, wb_ref[...])


def _node_update(h, aggs, Wnh, Wna, bn, Wa, Wb):
    return pl.pallas_call(
        _node_update_body,
        out_shape=[jax.ShapeDtypeStruct((N, L), _f32)] * 3,
    )(h, aggs, Wnh, Wna, bn, Wa, Wb)


def _final_body(h_ref, a_ref, wnh_ref, wna_ref, bn_ref, wd1_ref, bd1_ref,
                wd2_ref, bd2_ref, wr_ref, br_ref, out_ref):
    agg = a_ref[0] + a_ref[1]
    h2 = jnp.maximum(
        _dot(h_ref[...], wnh_ref[...]) + _dot(agg, wna_ref[...]) + bn_ref[...],
        0.0)
    d = jnp.maximum(_dot(h2, wd1_ref[...]) + bd1_ref[...], 0.0)
    d = jnp.maximum(_dot(d, wd2_ref[...]) + bd2_ref[...], 0.0)
    out_ref[...] = _dot(d, wr_ref[...]) + br_ref[...]


def _final(h, aggs, Wnh, Wna, bn, W_d1, b_d1, W_d2, b_d2, W_r, b_r):
    return pl.pallas_call(
        _final_body,
        out_shape=jax.ShapeDtypeStruct((N, 1), _f32),
    )(h, aggs, Wnh, Wna, bn, W_d1, b_d1, W_d2, b_d2, W_r, b_r)


def _make_sc_step(write_e: bool):
    mesh = plsc.VectorSubcoreMesh(core_axis_name="c", subcore_axis_name="s")
    out_type = [jax.ShapeDtypeStruct((NC, N, L), _f32)]
    if write_e:
        out_type = [jax.ShapeDtypeStruct((E, L), _f32)] + out_type

    @functools.partial(
        pl.kernel,
        mesh=mesh,
        out_type=out_type,
        scratch_types=[
            pltpu.VMEM((2, C), jnp.int32),
            pltpu.VMEM((2, C), jnp.int32),
            pltpu.VMEM((2, C), jnp.int32),
            pltpu.VMEM((2, C, L), _f32),
            pltpu.VMEM((2, C, L), _f32),
            pltpu.VMEM((2, C, L), jnp.int32),
            pltpu.VMEM((2, C, L), _f32),
            pltpu.VMEM_SHARED((N, L), _f32),
            pltpu.SemaphoreType.DMA,
            pltpu.SemaphoreType.DMA,
            pltpu.SemaphoreType.DMA,
            pltpu.SemaphoreType.DMA,
            pltpu.SemaphoreType.DMA,
            pltpu.SemaphoreType.DMA,
            pltpu.SemaphoreType.DMA,
            pltpu.SemaphoreType.DMA,
        ],
    )
    def sc_step(p_hbm, q_hbm, u_hbm, src_hbm, dst_hbm, *refs):
        if write_e:
            (e_out, agg_out, idx_s, idx_d, idx_c, buf_p, buf_q, buf_u, buf_e,
             agg_sh, sem_is, sem_id, sem_ic, sem_gp, sem_gq, sem_u, sem_we,
             sem_sc) = refs
        else:
            (agg_out, idx_s, idx_d, idx_c, buf_p, buf_q, buf_u, buf_e,
             agg_sh, sem_is, sem_id, sem_ic, sem_gp, sem_gq, sem_u, sem_we,
             sem_sc) = refs
        cid = lax.axis_index("c")
        sid = lax.axis_index("s")
        wid = sid * NC + cid
        base = wid * EPW

        def zfill(i, carry):
            for j in range(L // 16):
                buf_p[0, i, pl.ds(j * 16, 16)] = jnp.zeros((16,), _f32)
            return carry
        lax.fori_loop(0, C, zfill, 0)
        zbase = sid * RPS
        for z in range(RPS // C):
            pltpu.sync_copy(buf_p.at[0],
                            agg_sh.at[pl.ds(zbase + z * C, C)])
        if RPS % C:
            pltpu.sync_copy(buf_p.at[0, pl.ds(0, RPS % C)],
                            agg_sh.at[pl.ds(zbase + (RPS // C) * C, RPS % C)])

        @pl.when(sid == NS - 1)
        def _zero_tail():
            pltpu.sync_copy(buf_p.at[0, pl.ds(0, 16)],
                            agg_sh.at[pl.ds(NS * RPS, 16)])
        plsc.subcore_barrier()

        def issue_idx(k, slot):
            estart = base + k * C
            pltpu.async_copy(src_hbm.at[pl.ds(estart, C)],
                             idx_s.at[slot], sem_is)
            pltpu.async_copy(dst_hbm.at[pl.ds(estart, C)],
                             idx_d.at[slot], sem_id)

        def issue_idx_c(k, slot):
            pltpu.async_copy(dst_hbm.at[pl.ds(base + k * C, C)],
                             idx_c.at[slot], sem_ic)

        def wait_idx_c(slot):
            pltpu.make_async_copy(dst_hbm.at[pl.ds(0, C)],
                                  idx_c.at[slot], sem_ic).wait()

        def issue_u(p, uslot):
            off = pl.multiple_of(base // 2 + p * C, 8)
            pltpu.async_copy(u_hbm.at[pl.ds(off, C)],
                             buf_u.at[uslot], sem_u)

        def wait_u(uslot):
            pltpu.make_async_copy(u_hbm.at[pl.ds(0, C)],
                                  buf_u.at[uslot], sem_u).wait()

        def wait_idx(slot):
            pltpu.make_async_copy(src_hbm.at[pl.ds(0, C)],
                                  idx_s.at[slot], sem_is).wait()
            pltpu.make_async_copy(dst_hbm.at[pl.ds(0, C)],
                                  idx_d.at[slot], sem_id).wait()

        def issue_gathers(slot):
            pltpu.async_copy(p_hbm.at[idx_s.at[slot]], buf_p.at[slot], sem_gp)
            pltpu.async_copy(q_hbm.at[idx_d.at[slot]], buf_q.at[slot], sem_gq)

        def wait_gathers(slot):
            pltpu.make_async_copy(p_hbm.at[pl.ds(0, C)],
                                  buf_p.at[slot], sem_gp).wait()
            pltpu.make_async_copy(q_hbm.at[pl.ds(0, C)],
                                  buf_q.at[slot], sem_gq).wait()

        def wait_scatter(slot):
            pltpu.make_async_copy(buf_e.at[slot],
                                  agg_sh.at[pl.ds(0, C)], sem_sc).wait()

        def wait_ewrite(slot):
            if write_e:
                pltpu.make_async_copy(buf_e.at[slot],
                                      e_out.at[pl.ds(0, C)], sem_we).wait()

        issue_idx(0, 0)
        issue_idx(1, 1)
        issue_idx_c(0, 0)
        issue_u(0, 0)
        wait_idx(0)
        issue_gathers(0)

        def _maybe(cond, fn):
            if cond is None:
                fn()
            else:
                pl.when(cond)(fn)

        def do_chunk(k, slot, first, pref1, pref2, uslot):
            oslot = 1 - slot
            uoff = slot * (C // 2)
            if not first:
                wait_scatter(oslot)

            _maybe(pref1, lambda: issue_idx_c(k + 1, oslot))

            wait_gathers(slot)

            def _start_next_gathers():
                wait_idx(oslot)
                issue_gathers(oslot)
            _maybe(pref1, _start_next_gathers)
            _maybe(pref2, lambda: issue_idx(k + 2, slot))

            def row2(i2, rcarry):
                for j in range(L // 16):
                    s = pl.ds(j * 16, 16)
                    w = buf_u[uslot, uoff + i2, s]
                    ulo = jax.lax.bitcast_convert_type(
                        jax.lax.shift_left(w, jnp.int32(16)), _f32)
                    uhi = jax.lax.bitcast_convert_type(
                        jnp.bitwise_and(w, jnp.int32(-65536)), _f32)
                    e0 = (buf_p[slot, 2 * i2, s] + buf_q[slot, 2 * i2, s]
                          + ulo)
                    e1 = (buf_p[slot, 2 * i2 + 1, s]
                          + buf_q[slot, 2 * i2 + 1, s] + uhi)
                    buf_e[slot, 2 * i2, s] = jnp.maximum(e0, 0.0)
                    buf_e[slot, 2 * i2 + 1, s] = jnp.maximum(e1, 0.0)
                return rcarry
            lax.fori_loop(0, C // 2, row2, 0)

            if not first:
                wait_ewrite(oslot)
            wait_idx_c(slot)
            estart = base + k * C
            if write_e:
                pltpu.async_copy(buf_e.at[slot],
                                 e_out.at[pl.ds(estart, C)], sem_we)
            pltpu.async_copy(buf_e.at[slot],
                             agg_sh.at[idx_c.at[slot]], sem_sc, add=True)

        NQUAD = (NCHUNK - 2) // 4

        def quad(q, carry):
            p1 = 1 + 2 * q
            p2 = 2 + 2 * q
            issue_u(p1 + 1, 0)
            wait_u(1)
            do_chunk(2 * p1, 0, first=False, pref1=None, pref2=None, uslot=1)
            do_chunk(2 * p1 + 1, 1, first=False, pref1=None, pref2=None,
                     uslot=1)
            pl.when(q < NQUAD - 1)(lambda: issue_u(p2 + 1, 1))
            wait_u(0)
            do_chunk(2 * p2, 0, first=False, pref1=None,
                     pref2=(q < NQUAD - 1), uslot=0)
            do_chunk(2 * p2 + 1, 1, first=False, pref1=(q < NQUAD - 1),
                     pref2=(q < NQUAD - 1), uslot=0)
            return carry

        issue_u(1, 1)
        wait_u(0)
        do_chunk(0, 0, first=True, pref1=None, pref2=None, uslot=0)
        do_chunk(1, 1, first=False, pref1=None, pref2=None, uslot=0)
        lax.fori_loop(0, NQUAD, quad, 0)
        wait_scatter(1)
        wait_ewrite(1)

        plsc.subcore_barrier()
        pltpu.sync_copy(agg_sh.at[pl.ds(sid * RPS, RPS)],
                        agg_out.at[cid, pl.ds(sid * RPS, RPS)])

        @pl.when(sid == NS - 1)
        def _copy_tail():
            pltpu.sync_copy(agg_sh.at[pl.ds(NS * RPS, 16)],
                            agg_out.at[cid, pl.ds(NS * RPS, 16)])

    return sc_step


_sc_step_we = _make_sc_step(write_e=True)
_sc_step_ne = _make_sc_step(write_e=False)


def kernel(x, edge_index, edge_attr, W_ne, b_ne, W_ee, b_ee, W_e, b_e,
           W_n, b_n, W_d1, b_d1, W_d2, b_d2, W_r, b_r):
    src = edge_index[0].astype(jnp.int32)
    dst = edge_index[1].astype(jnp.int32)

    Wa0, Wb0, Wc0 = W_e[0, :L], W_e[0, L:2 * L], W_e[0, 2 * L:]
    Wa1, Wb1, Wc1 = W_e[1, :L], W_e[1, L:2 * L], W_e[1, 2 * L:]
    Wn0h, Wn0a = W_n[0, :L], W_n[0, L:]
    Wn1h, Wn1a = W_n[1, :L], W_n[1, L:]
    bne = b_ne.reshape(1, L)
    bee = b_ee.reshape(1, L)
    be0 = b_e[0].reshape(1, L)
    be1 = b_e[1].reshape(1, L)
    bn0 = b_n[0].reshape(1, L)
    bn1 = b_n[1].reshape(1, L)
    bd1 = b_d1.reshape(1, L)
    bd2 = b_d2.reshape(1, L)
    br = b_r.reshape(1, 1)

    h0, P0, Q0 = _node_encode(x, W_ne, bne, Wa0, Wb0)
    U0 = _edge_u0(edge_attr, W_ee, bee, Wc0, be0)
    e1, agg0 = _sc_step_we(P0, Q0, U0, src, dst)
    h1, P1, Q1 = _node_update(h0, agg0, Wn0h, Wn0a, bn0, Wa1, Wb1)
    U1 = _edge_u(e1, Wc1, be1)
    (agg1,) = _sc_step_ne(P1, Q1, U1, src, dst)
    out = _final(h1, agg1, Wn1h, Wn1a, bn1, W_d1, bd1, W_d2, bd2, W_r, br)
    return out


# speedup vs baseline: 1.3532x; 1.3532x over previous
import functools

import jax
import jax.numpy as jnp
from jax import lax
from jax.experimental import pallas as pl
from jax.experimental.pallas import tpu as pltpu
from jax.experimental.pallas import tpu_sc as plsc

N = 10000
E = 320000
DF = 128
DE = 4
L = 128

NC = 2
NS = 16
NW = NC * NS
EH = E // 2
EPW = EH // NW
C = 40
NCHUNK = EPW // C
RPS = 624

_f32 = jnp.float32


def _dot(a, b):
    return jnp.dot(a, b, preferred_element_type=_f32)


def _dot16(a, b):
    return jnp.dot(a.astype(jnp.bfloat16), b.astype(jnp.bfloat16),
                   preferred_element_type=_f32)


def _node_encode_body(x_ref, wne_ref, bne_ref, wa_ref, wb_ref,
                      h_ref, p_ref, q_ref):
    h = jnp.maximum(_dot(x_ref[...], wne_ref[...]) + bne_ref[...], 0.0)
    h_ref[...] = h
    p_ref[...] = _dot(h, wa_ref[...])
    q_ref[...] = _dot(h, wb_ref[...])


def _node_encode(x, W_ne, b_ne, Wa, Wb):
    return pl.pallas_call(
        _node_encode_body,
        out_shape=[jax.ShapeDtypeStruct((N, L), _f32)] * 3,
    )(x, W_ne, b_ne, Wa, Wb)


BE = 6400


def _edge_u0_body(ea_ref, wee_ref, bee_ref, wc_ref, be_ref, u_ref):
    e0 = jnp.maximum(_dot(ea_ref[...], wee_ref[...]) + bee_ref[...], 0.0)
    u_ref[...] = _dot16(e0, wc_ref[...]) + be_ref[...]


def _edge_u0(edge_attr, W_ee, b_ee, Wc, be):
    rows = edge_attr.shape[0]
    return pl.pallas_call(
        _edge_u0_body,
        grid=(rows // BE,),
        in_specs=[
            pl.BlockSpec((BE, DE), lambda i: (i, 0)),
            pl.BlockSpec((DE, L), lambda i: (0, 0)),
            pl.BlockSpec((1, L), lambda i: (0, 0)),
            pl.BlockSpec((L, L), lambda i: (0, 0)),
            pl.BlockSpec((1, L), lambda i: (0, 0)),
        ],
        out_specs=pl.BlockSpec((BE, L), lambda i: (i, 0)),
        out_shape=jax.ShapeDtypeStruct((rows, L), _f32),
    )(edge_attr, W_ee, b_ee, Wc, be)


def _edge_u_body(e_ref, wc_ref, be_ref, u_ref):
    u_ref[...] = _dot16(e_ref[...], wc_ref[...]) + be_ref[...]


def _edge_u(e, Wc, be):
    rows = e.shape[0]
    return pl.pallas_call(
        _edge_u_body,
        grid=(rows // BE,),
        in_specs=[
            pl.BlockSpec((BE, L), lambda i: (i, 0)),
            pl.BlockSpec((L, L), lambda i: (0, 0)),
            pl.BlockSpec((1, L), lambda i: (0, 0)),
        ],
        out_specs=pl.BlockSpec((BE, L), lambda i: (i, 0)),
        out_shape=jax.ShapeDtypeStruct((rows, L), _f32),
    )(e, Wc, be)


def _node_update_body(h_ref, a_ref, b_ref2, wnh_ref, wna_ref, bn_ref,
                      wa_ref, wb_ref, h1_ref, p_ref, q_ref):
    agg = (a_ref[0] + a_ref[1]) + (b_ref2[0] + b_ref2[1])
    h1 = jnp.maximum(
        _dot(h_ref[...], wnh_ref[...]) + _dot(agg, wna_ref[...]) + bn_ref[...],
        0.0)
    h1_ref[...] = h1
    p_ref[...] = _dot(h1, wa_ref[...])
    q_ref[...] = _dot(h1, wb_ref[...])


def _node_update(h, agg_a, agg_b, Wnh, Wna, bn, Wa, Wb):
    return pl.pallas_call(
        _node_update_body,
        out_shape=[jax.ShapeDtypeStruct((N, L), _f32)] * 3,
    )(h, agg_a, agg_b, Wnh, Wna, bn, Wa, Wb)


def _final_body(h_ref, a_ref, b_ref2, wnh_ref, wna_ref, bn_ref, wd1_ref,
                bd1_ref, wd2_ref, bd2_ref, wr_ref, br_ref, out_ref):
    agg = (a_ref[0] + a_ref[1]) + (b_ref2[0] + b_ref2[1])
    h2 = jnp.maximum(
        _dot(h_ref[...], wnh_ref[...]) + _dot(agg, wna_ref[...]) + bn_ref[...],
        0.0)
    d = jnp.maximum(_dot(h2, wd1_ref[...]) + bd1_ref[...], 0.0)
    d = jnp.maximum(_dot(d, wd2_ref[...]) + bd2_ref[...], 0.0)
    out_ref[...] = _dot(d, wr_ref[...]) + br_ref[...]


def _final(h, agg_a, agg_b, Wnh, Wna, bn, W_d1, b_d1, W_d2, b_d2, W_r, b_r):
    return pl.pallas_call(
        _final_body,
        out_shape=jax.ShapeDtypeStruct((N, 1), _f32),
    )(h, agg_a, agg_b, Wnh, Wna, bn, W_d1, b_d1, W_d2, b_d2, W_r, b_r)


def _make_sc_step(write_e: bool):
    mesh = plsc.VectorSubcoreMesh(core_axis_name="c", subcore_axis_name="s")
    out_type = [jax.ShapeDtypeStruct((NC, N, L), _f32)]
    if write_e:
        out_type = [jax.ShapeDtypeStruct((EH, L), _f32)] + out_type

    @functools.partial(
        pl.kernel,
        mesh=mesh,
        out_type=out_type,
        scratch_types=[
            pltpu.VMEM((2, C), jnp.int32),
            pltpu.VMEM((2, C), jnp.int32),
            pltpu.VMEM((2, C), jnp.int32),
            pltpu.VMEM((2, C, L), _f32),
            pltpu.VMEM((2, C, L), _f32),
            pltpu.VMEM((2, C, L), _f32),
            pltpu.VMEM((2, C, L), _f32),
            pltpu.VMEM_SHARED((N, L), _f32),
            pltpu.SemaphoreType.DMA,
            pltpu.SemaphoreType.DMA,
            pltpu.SemaphoreType.DMA,
            pltpu.SemaphoreType.DMA,
            pltpu.SemaphoreType.DMA,
            pltpu.SemaphoreType.DMA,
            pltpu.SemaphoreType.DMA,
            pltpu.SemaphoreType.DMA,
        ],
    )
    def sc_step(p_hbm, q_hbm, u_hbm, src_hbm, dst_hbm, *refs):
        if write_e:
            (e_out, agg_out, idx_s, idx_d, idx_c, buf_p, buf_q, buf_u, buf_e,
             agg_sh, sem_is, sem_id, sem_ic, sem_gp, sem_gq, sem_u, sem_we,
             sem_sc) = refs
        else:
            (agg_out, idx_s, idx_d, idx_c, buf_p, buf_q, buf_u, buf_e,
             agg_sh, sem_is, sem_id, sem_ic, sem_gp, sem_gq, sem_u, sem_we,
             sem_sc) = refs
        cid = lax.axis_index("c")
        sid = lax.axis_index("s")
        wid = sid * NC + cid
        base = wid * EPW

        def zfill(i, carry):
            for j in range(L // 16):
                buf_p[0, i, pl.ds(j * 16, 16)] = jnp.zeros((16,), _f32)
            return carry
        lax.fori_loop(0, C, zfill, 0)
        zbase = sid * RPS
        for z in range(RPS // C):
            pltpu.async_copy(buf_p.at[0],
                             agg_sh.at[pl.ds(zbase + z * C, C)], sem_we)
        if RPS % C:
            pltpu.async_copy(buf_p.at[0, pl.ds(0, RPS % C)],
                             agg_sh.at[pl.ds(zbase + (RPS // C) * C, RPS % C)],
                             sem_we)

        @pl.when(sid == NS - 1)
        def _zero_tail():
            pltpu.async_copy(buf_p.at[0, pl.ds(0, 16)],
                             agg_sh.at[pl.ds(NS * RPS, 16)], sem_we)
        for z in range(RPS // C):
            pltpu.make_async_copy(buf_p.at[0],
                                  agg_sh.at[pl.ds(0, C)], sem_we).wait()
        if RPS % C:
            pltpu.make_async_copy(buf_p.at[0, pl.ds(0, RPS % C)],
                                  agg_sh.at[pl.ds(0, RPS % C)], sem_we).wait()

        @pl.when(sid == NS - 1)
        def _zero_tail_wait():
            pltpu.make_async_copy(buf_p.at[0, pl.ds(0, 16)],
                                  agg_sh.at[pl.ds(0, 16)], sem_we).wait()
        plsc.subcore_barrier()

        def issue_idx(k, slot):
            estart = base + k * C
            pltpu.async_copy(src_hbm.at[pl.ds(estart, C)],
                             idx_s.at[slot], sem_is)
            pltpu.async_copy(dst_hbm.at[pl.ds(estart, C)],
                             idx_d.at[slot], sem_id)

        def issue_idx_c(k, slot):
            pltpu.async_copy(dst_hbm.at[pl.ds(base + k * C, C)],
                             idx_c.at[slot], sem_ic)

        def wait_idx_c(slot):
            pltpu.make_async_copy(dst_hbm.at[pl.ds(0, C)],
                                  idx_c.at[slot], sem_ic).wait()

        def issue_u(k, slot):
            pltpu.async_copy(u_hbm.at[pl.ds(base + k * C, C)],
                             buf_u.at[slot], sem_u)

        def wait_idx(slot):
            pltpu.make_async_copy(src_hbm.at[pl.ds(0, C)],
                                  idx_s.at[slot], sem_is).wait()
            pltpu.make_async_copy(dst_hbm.at[pl.ds(0, C)],
                                  idx_d.at[slot], sem_id).wait()

        def issue_gathers(slot):
            pltpu.async_copy(p_hbm.at[idx_s.at[slot]], buf_p.at[slot], sem_gp)
            pltpu.async_copy(q_hbm.at[idx_d.at[slot]], buf_q.at[slot], sem_gq)

        def wait_gathers_u(slot):
            pltpu.make_async_copy(p_hbm.at[pl.ds(0, C)],
                                  buf_p.at[slot], sem_gp).wait()
            pltpu.make_async_copy(q_hbm.at[pl.ds(0, C)],
                                  buf_q.at[slot], sem_gq).wait()
            pltpu.make_async_copy(u_hbm.at[pl.ds(0, C)],
                                  buf_u.at[slot], sem_u).wait()

        def wait_scatter(slot):
            pltpu.make_async_copy(buf_e.at[slot],
                                  agg_sh.at[pl.ds(0, C)], sem_sc).wait()

        def wait_ewrite(slot):
            if write_e:
                pltpu.make_async_copy(buf_e.at[slot],
                                      e_out.at[pl.ds(0, C)], sem_we).wait()

        issue_idx(0, 0)
        issue_idx(1, 1)
        issue_idx_c(0, 0)
        issue_u(0, 0)
        wait_idx(0)
        issue_gathers(0)

        def _maybe(cond, fn):
            if cond is None:
                fn()
            else:
                pl.when(cond)(fn)

        def do_chunk(k, slot, first, pref1, pref2):
            oslot = 1 - slot
            if not first:
                wait_scatter(oslot)

            def _prefetch_ucn():
                issue_u(k + 1, oslot)
                issue_idx_c(k + 1, oslot)
            _maybe(pref1, _prefetch_ucn)

            wait_gathers_u(slot)

            def _start_next_gathers():
                wait_idx(oslot)
                issue_gathers(oslot)
            _maybe(pref1, _start_next_gathers)
            _maybe(pref2, lambda: issue_idx(k + 2, slot))

            def row(i, rcarry):
                for j in range(L // 16):
                    s = pl.ds(j * 16, 16)
                    v = (buf_p[slot, i, s] + buf_q[slot, i, s]
                         + buf_u[slot, i, s])
                    buf_e[slot, i, s] = jnp.maximum(v, 0.0)
                return rcarry
            lax.fori_loop(0, C, row, 0)

            if not first:
                wait_ewrite(oslot)
            wait_idx_c(slot)
            estart = base + k * C
            if write_e:
                pltpu.async_copy(buf_e.at[slot],
                                 e_out.at[pl.ds(estart, C)], sem_we)
            pltpu.async_copy(buf_e.at[slot],
                             agg_sh.at[idx_c.at[slot]], sem_sc, add=True)

        NPAIR = (NCHUNK - 1) // 2

        def pair(t, carry):
            do_chunk(2 * t - 1, 1, first=False, pref1=None,
                     pref2=(t < NPAIR))
            do_chunk(2 * t, 0, first=False, pref1=(t < NPAIR),
                     pref2=(t < NPAIR))
            return carry

        do_chunk(0, 0, first=True, pref1=None, pref2=None)
        lax.fori_loop(1, NPAIR + 1, pair, 0)
        wait_scatter((NCHUNK - 1) % 2)
        wait_ewrite((NCHUNK - 1) % 2)

        plsc.subcore_barrier()
        pltpu.sync_copy(agg_sh.at[pl.ds(sid * RPS, RPS)],
                        agg_out.at[cid, pl.ds(sid * RPS, RPS)])

        @pl.when(sid == NS - 1)
        def _copy_tail():
            pltpu.sync_copy(agg_sh.at[pl.ds(NS * RPS, 16)],
                            agg_out.at[cid, pl.ds(NS * RPS, 16)])

    return sc_step


_sc_step_we = _make_sc_step(write_e=True)
_sc_step_ne = _make_sc_step(write_e=False)


def kernel(x, edge_index, edge_attr, W_ne, b_ne, W_ee, b_ee, W_e, b_e,
           W_n, b_n, W_d1, b_d1, W_d2, b_d2, W_r, b_r):
    src = edge_index[0].astype(jnp.int32)
    dst = edge_index[1].astype(jnp.int32)

    Wa0, Wb0, Wc0 = W_e[0, :L], W_e[0, L:2 * L], W_e[0, 2 * L:]
    Wa1, Wb1, Wc1 = W_e[1, :L], W_e[1, L:2 * L], W_e[1, 2 * L:]
    Wn0h, Wn0a = W_n[0, :L], W_n[0, L:]
    Wn1h, Wn1a = W_n[1, :L], W_n[1, L:]
    bne = b_ne.reshape(1, L)
    bee = b_ee.reshape(1, L)
    be0 = b_e[0].reshape(1, L)
    be1 = b_e[1].reshape(1, L)
    bn0 = b_n[0].reshape(1, L)
    bn1 = b_n[1].reshape(1, L)
    bd1 = b_d1.reshape(1, L)
    bd2 = b_d2.reshape(1, L)
    br = b_r.reshape(1, 1)

    src_a, src_b = src[:EH], src[EH:]
    dst_a, dst_b = dst[:EH], dst[EH:]

    h0, P0, Q0 = _node_encode(x, W_ne, bne, Wa0, Wb0)
    U0a = _edge_u0(edge_attr[:EH], W_ee, bee, Wc0, be0)
    U0b = _edge_u0(edge_attr[EH:], W_ee, bee, Wc0, be0)
    e1a, agg0a = _sc_step_we(P0, Q0, U0a, src_a, dst_a)
    e1b, agg0b = _sc_step_we(P0, Q0, U0b, src_b, dst_b)
    U1a = _edge_u(e1a, Wc1, be1)
    U1b = _edge_u(e1b, Wc1, be1)
    h1, P1, Q1 = _node_update(h0, agg0a, agg0b, Wn0h, Wn0a, bn0, Wa1, Wb1)
    (agg1a,) = _sc_step_ne(P1, Q1, U1a, src_a, dst_a)
    (agg1b,) = _sc_step_ne(P1, Q1, U1b, src_b, dst_b)
    out = _final(h1, agg1a, agg1b, Wn1h, Wn1a, bn1,
                 W_d1, bd1, W_d2, bd2, W_r, br)
    return out


# speedup vs baseline: 1.3601x; 1.0051x over previous
import functools

import jax
import jax.numpy as jnp
from jax import lax
from jax.experimental import pallas as pl
from jax.experimental.pallas import tpu as pltpu
from jax.experimental.pallas import tpu_sc as plsc

N = 10000
E = 320000
DF = 128
DE = 4
L = 128

NC = 2
NS = 16
NW = NC * NS
EH = E // 2
EPW = EH // NW
C = 40
NCHUNK = EPW // C
RPS = 624

_f32 = jnp.float32


def _dot(a, b):
    return jnp.dot(a, b, preferred_element_type=_f32)


def _dot16(a, b):
    return jnp.dot(a.astype(jnp.bfloat16), b.astype(jnp.bfloat16),
                   preferred_element_type=_f32)


def _node_encode_body(x_ref, wne_ref, bne_ref, wa_ref, wb_ref,
                      h_ref, p_ref, q_ref):
    h = jnp.maximum(_dot(x_ref[...], wne_ref[...]) + bne_ref[...], 0.0)
    h_ref[...] = h
    p_ref[...] = _dot(h, wa_ref[...])
    q_ref[...] = _dot(h, wb_ref[...])


def _node_encode(x, W_ne, b_ne, Wa, Wb):
    return pl.pallas_call(
        _node_encode_body,
        out_shape=[jax.ShapeDtypeStruct((N, L), _f32)] * 3,
    )(x, W_ne, b_ne, Wa, Wb)


BE = 6400


def _edge_u0_body(ea_ref, wee_ref, bee_ref, wc_ref, be_ref, u_ref):
    e0 = jnp.maximum(_dot(ea_ref[...], wee_ref[...]) + bee_ref[...], 0.0)
    u_ref[...] = _dot16(e0, wc_ref[...]) + be_ref[...]


def _edge_u0(edge_attr, W_ee, b_ee, Wc, be):
    rows = edge_attr.shape[0]
    return pl.pallas_call(
        _edge_u0_body,
        grid=(rows // BE,),
        in_specs=[
            pl.BlockSpec((BE, DE), lambda i: (i, 0)),
            pl.BlockSpec((DE, L), lambda i: (0, 0)),
            pl.BlockSpec((1, L), lambda i: (0, 0)),
            pl.BlockSpec((L, L), lambda i: (0, 0)),
            pl.BlockSpec((1, L), lambda i: (0, 0)),
        ],
        out_specs=pl.BlockSpec((BE, L), lambda i: (i, 0)),
        out_shape=jax.ShapeDtypeStruct((rows, L), _f32),
    )(edge_attr, W_ee, b_ee, Wc, be)


def _edge_u_body(e_ref, wc_ref, be_ref, u_ref):
    u_ref[...] = _dot16(e_ref[...], wc_ref[...]) + be_ref[...]


def _edge_u(e, Wc, be):
    rows = e.shape[0]
    return pl.pallas_call(
        _edge_u_body,
        grid=(rows // BE,),
        in_specs=[
            pl.BlockSpec((BE, L), lambda i: (i, 0)),
            pl.BlockSpec((L, L), lambda i: (0, 0)),
            pl.BlockSpec((1, L), lambda i: (0, 0)),
        ],
        out_specs=pl.BlockSpec((BE, L), lambda i: (i, 0)),
        out_shape=jax.ShapeDtypeStruct((rows, L), _f32),
    )(e, Wc, be)


def _node_update_body(h_ref, a_ref, b_ref2, wnh_ref, wna_ref, bn_ref,
                      wa_ref, wb_ref, h1_ref, p_ref, q_ref):
    agg = (a_ref[0] + a_ref[1]) + (b_ref2[0] + b_ref2[1])
    h1 = jnp.maximum(
        _dot(h_ref[...], wnh_ref[...]) + _dot(agg, wna_ref[...]) + bn_ref[...],
        0.0)
    h1_ref[...] = h1
    p_ref[...] = _dot(h1, wa_ref[...])
    q_ref[...] = _dot(h1, wb_ref[...])


def _node_update(h, agg_a, agg_b, Wnh, Wna, bn, Wa, Wb):
    return pl.pallas_call(
        _node_update_body,
        out_shape=[jax.ShapeDtypeStruct((N, L), _f32)] * 3,
    )(h, agg_a, agg_b, Wnh, Wna, bn, Wa, Wb)


def _final_body(h_ref, a_ref, b_ref2, wnh_ref, wna_ref, bn_ref, wd1_ref,
                bd1_ref, wd2_ref, bd2_ref, wr_ref, br_ref, out_ref):
    agg = (a_ref[0] + a_ref[1]) + (b_ref2[0] + b_ref2[1])
    h2 = jnp.maximum(
        _dot(h_ref[...], wnh_ref[...]) + _dot(agg, wna_ref[...]) + bn_ref[...],
        0.0)
    d = jnp.maximum(_dot(h2, wd1_ref[...]) + bd1_ref[...], 0.0)
    d = jnp.maximum(_dot(d, wd2_ref[...]) + bd2_ref[...], 0.0)
    out_ref[...] = _dot(d, wr_ref[...]) + br_ref[...]


def _final(h, agg_a, agg_b, Wnh, Wna, bn, W_d1, b_d1, W_d2, b_d2, W_r, b_r):
    return pl.pallas_call(
        _final_body,
        out_shape=jax.ShapeDtypeStruct((N, 1), _f32),
    )(h, agg_a, agg_b, Wnh, Wna, bn, W_d1, b_d1, W_d2, b_d2, W_r, b_r)


def _make_sc_step(write_e: bool):
    mesh = plsc.VectorSubcoreMesh(core_axis_name="c", subcore_axis_name="s")
    out_type = [jax.ShapeDtypeStruct((NC, N, L), _f32)]
    if write_e:
        out_type = [jax.ShapeDtypeStruct((EH, L), _f32)] + out_type

    @functools.partial(
        pl.kernel,
        mesh=mesh,
        out_type=out_type,
        scratch_types=[
            pltpu.VMEM((2, C), jnp.int32),
            pltpu.VMEM((2, C), jnp.int32),
            pltpu.VMEM((2, C), jnp.int32),
            pltpu.VMEM((2, C, L), _f32),
            pltpu.VMEM((2, C, L), _f32),
            pltpu.VMEM((2, C, L), _f32),
            pltpu.VMEM((2, C, L), _f32),
            pltpu.VMEM_SHARED((N, L), _f32),
            pltpu.SemaphoreType.DMA,
            pltpu.SemaphoreType.DMA,
            pltpu.SemaphoreType.DMA,
            pltpu.SemaphoreType.DMA,
            pltpu.SemaphoreType.DMA,
            pltpu.SemaphoreType.DMA,
            pltpu.SemaphoreType.DMA,
            pltpu.SemaphoreType.DMA,
        ],
    )
    def sc_step(p_hbm, q_hbm, u_hbm, src_hbm, dst_hbm, *refs):
        if write_e:
            (e_out, agg_out, idx_s, idx_d, idx_c, buf_p, buf_q, buf_u, buf_e,
             agg_sh, sem_is, sem_id, sem_ic, sem_gp, sem_gq, sem_u, sem_we,
             sem_sc) = refs
        else:
            (agg_out, idx_s, idx_d, idx_c, buf_p, buf_q, buf_u, buf_e,
             agg_sh, sem_is, sem_id, sem_ic, sem_gp, sem_gq, sem_u, sem_we,
             sem_sc) = refs
        cid = lax.axis_index("c")
        sid = lax.axis_index("s")
        wid = sid * NC + cid
        base = wid * EPW

        def issue_idx(k, slot):
            estart = base + k * C
            pltpu.async_copy(src_hbm.at[pl.ds(estart, C)],
                             idx_s.at[slot], sem_is)
            pltpu.async_copy(dst_hbm.at[pl.ds(estart, C)],
                             idx_d.at[slot], sem_id)

        def issue_idx_c(k, slot):
            pltpu.async_copy(dst_hbm.at[pl.ds(base + k * C, C)],
                             idx_c.at[slot], sem_ic)

        def wait_idx_c(slot):
            pltpu.make_async_copy(dst_hbm.at[pl.ds(0, C)],
                                  idx_c.at[slot], sem_ic).wait()

        def issue_u(k, slot):
            pltpu.async_copy(u_hbm.at[pl.ds(base + k * C, C)],
                             buf_u.at[slot], sem_u)

        def wait_idx(slot):
            pltpu.make_async_copy(src_hbm.at[pl.ds(0, C)],
                                  idx_s.at[slot], sem_is).wait()
            pltpu.make_async_copy(dst_hbm.at[pl.ds(0, C)],
                                  idx_d.at[slot], sem_id).wait()

        def issue_gathers(slot):
            pltpu.async_copy(p_hbm.at[idx_s.at[slot]], buf_p.at[slot], sem_gp)
            pltpu.async_copy(q_hbm.at[idx_d.at[slot]], buf_q.at[slot], sem_gq)

        def wait_gathers_u(slot):
            pltpu.make_async_copy(p_hbm.at[pl.ds(0, C)],
                                  buf_p.at[slot], sem_gp).wait()
            pltpu.make_async_copy(q_hbm.at[pl.ds(0, C)],
                                  buf_q.at[slot], sem_gq).wait()
            pltpu.make_async_copy(u_hbm.at[pl.ds(0, C)],
                                  buf_u.at[slot], sem_u).wait()

        def wait_scatter(slot):
            pltpu.make_async_copy(buf_e.at[slot],
                                  agg_sh.at[pl.ds(0, C)], sem_sc).wait()

        def wait_ewrite(slot):
            if write_e:
                pltpu.make_async_copy(buf_e.at[slot],
                                      e_out.at[pl.ds(0, C)], sem_we).wait()

        issue_idx(0, 0)
        issue_idx(1, 1)
        issue_idx_c(0, 0)
        issue_u(0, 0)
        wait_idx(0)
        issue_gathers(0)

        def zfill(i, carry):
            for j in range(L // 16):
                buf_e[0, i, pl.ds(j * 16, 16)] = jnp.zeros((16,), _f32)
            return carry
        lax.fori_loop(0, C, zfill, 0)
        zbase = sid * RPS
        for z in range(RPS // C):
            pltpu.async_copy(buf_e.at[0],
                             agg_sh.at[pl.ds(zbase + z * C, C)], sem_we)
        if RPS % C:
            pltpu.async_copy(buf_e.at[0, pl.ds(0, RPS % C)],
                             agg_sh.at[pl.ds(zbase + (RPS // C) * C, RPS % C)],
                             sem_we)

        @pl.when(sid == NS - 1)
        def _zero_tail():
            pltpu.async_copy(buf_e.at[0, pl.ds(0, 16)],
                             agg_sh.at[pl.ds(NS * RPS, 16)], sem_we)
        for z in range(RPS // C):
            pltpu.make_async_copy(buf_e.at[0],
                                  agg_sh.at[pl.ds(0, C)], sem_we).wait()
        if RPS % C:
            pltpu.make_async_copy(buf_e.at[0, pl.ds(0, RPS % C)],
                                  agg_sh.at[pl.ds(0, RPS % C)], sem_we).wait()

        @pl.when(sid == NS - 1)
        def _zero_tail_wait():
            pltpu.make_async_copy(buf_e.at[0, pl.ds(0, 16)],
                                  agg_sh.at[pl.ds(0, 16)], sem_we).wait()
        plsc.subcore_barrier()

        def _maybe(cond, fn):
            if cond is None:
                fn()
            else:
                pl.when(cond)(fn)

        def do_chunk(k, slot, first, pref1, pref2):
            oslot = 1 - slot
            if not first:
                wait_scatter(oslot)

            def _prefetch_ucn():
                issue_u(k + 1, oslot)
                issue_idx_c(k + 1, oslot)
            _maybe(pref1, _prefetch_ucn)

            wait_gathers_u(slot)

            def _start_next_gathers():
                wait_idx(oslot)
                issue_gathers(oslot)
            _maybe(pref1, _start_next_gathers)
            _maybe(pref2, lambda: issue_idx(k + 2, slot))

            def row(i, rcarry):
                for j in range(L // 16):
                    s = pl.ds(j * 16, 16)
                    v = (buf_p[slot, i, s] + buf_q[slot, i, s]
                         + buf_u[slot, i, s])
                    buf_e[slot, i, s] = jnp.maximum(v, 0.0)
                return rcarry
            lax.fori_loop(0, C, row, 0)

            if not first:
                wait_ewrite(oslot)
            wait_idx_c(slot)
            estart = base + k * C
            if write_e:
                pltpu.async_copy(buf_e.at[slot],
                                 e_out.at[pl.ds(estart, C)], sem_we)
            pltpu.async_copy(buf_e.at[slot],
                             agg_sh.at[idx_c.at[slot]], sem_sc, add=True)

        NPAIR = (NCHUNK - 1) // 2

        def pair(t, carry):
            do_chunk(2 * t - 1, 1, first=False, pref1=None,
                     pref2=(t < NPAIR))
            do_chunk(2 * t, 0, first=False, pref1=(t < NPAIR),
                     pref2=(t < NPAIR))
            return carry

        do_chunk(0, 0, first=True, pref1=None, pref2=None)
        lax.fori_loop(1, NPAIR + 1, pair, 0)
        wait_scatter((NCHUNK - 1) % 2)
        wait_ewrite((NCHUNK - 1) % 2)

        plsc.subcore_barrier()
        pltpu.sync_copy(agg_sh.at[pl.ds(sid * RPS, RPS)],
                        agg_out.at[cid, pl.ds(sid * RPS, RPS)])

        @pl.when(sid == NS - 1)
        def _copy_tail():
            pltpu.sync_copy(agg_sh.at[pl.ds(NS * RPS, 16)],
                            agg_out.at[cid, pl.ds(NS * RPS, 16)])

    return sc_step


_sc_step_we = _make_sc_step(write_e=True)
_sc_step_ne = _make_sc_step(write_e=False)


def kernel(x, edge_index, edge_attr, W_ne, b_ne, W_ee, b_ee, W_e, b_e,
           W_n, b_n, W_d1, b_d1, W_d2, b_d2, W_r, b_r):
    src = edge_index[0].astype(jnp.int32)
    dst = edge_index[1].astype(jnp.int32)

    Wa0, Wb0, Wc0 = W_e[0, :L], W_e[0, L:2 * L], W_e[0, 2 * L:]
    Wa1, Wb1, Wc1 = W_e[1, :L], W_e[1, L:2 * L], W_e[1, 2 * L:]
    Wn0h, Wn0a = W_n[0, :L], W_n[0, L:]
    Wn1h, Wn1a = W_n[1, :L], W_n[1, L:]
    bne = b_ne.reshape(1, L)
    bee = b_ee.reshape(1, L)
    be0 = b_e[0].reshape(1, L)
    be1 = b_e[1].reshape(1, L)
    bn0 = b_n[0].reshape(1, L)
    bn1 = b_n[1].reshape(1, L)
    bd1 = b_d1.reshape(1, L)
    bd2 = b_d2.reshape(1, L)
    br = b_r.reshape(1, 1)

    src_a, src_b = src[:EH], src[EH:]
    dst_a, dst_b = dst[:EH], dst[EH:]

    h0, P0, Q0 = _node_encode(x, W_ne, bne, Wa0, Wb0)
    U0a = _edge_u0(edge_attr[:EH], W_ee, bee, Wc0, be0)
    U0b = _edge_u0(edge_attr[EH:], W_ee, bee, Wc0, be0)
    e1a, agg0a = _sc_step_we(P0, Q0, U0a, src_a, dst_a)
    e1b, agg0b = _sc_step_we(P0, Q0, U0b, src_b, dst_b)
    U1a = _edge_u(e1a, Wc1, be1)
    U1b = _edge_u(e1b, Wc1, be1)
    h1, P1, Q1 = _node_update(h0, agg0a, agg0b, Wn0h, Wn0a, bn0, Wa1, Wb1)
    (agg1a,) = _sc_step_ne(P1, Q1, U1a, src_a, dst_a)
    (agg1b,) = _sc_step_ne(P1, Q1, U1b, src_b, dst_b)
    out = _final(h1, agg1a, agg1b, Wn1h, Wn1a, bn1,
                 W_d1, bd1, W_d2, bd2, W_r, br)
    return out
